# baseline (speedup 1.0000x reference)
import functools

import jax
import jax.numpy as jnp
from jax import lax
from jax.experimental import pallas as pl
from jax.experimental.pallas import tpu as pltpu
from jax.experimental.pallas import tpu_sc as plsc

N_CLS = 1000
N_CTX = 16
CTX_DIM = 512
BATCH = 4096
SUF_LEN = 60
TOT_LEN = 1 + N_CTX + SUF_LEN

_D = CTX_DIM
_CTX_W = N_CTX * _D
_SUF_W = SUF_LEN * _D
_OUT_W = TOT_LEN * _D

_NC = 2
_NS = 16
_NW = _NC * _NS
_BPW = BATCH // _NW
_CH = 8
_NCHUNK = _BPW // _CH


def _sc_prompt_kernel():
    mesh = plsc.VectorSubcoreMesh(core_axis_name="c", subcore_axis_name="s")

    @functools.partial(
        pl.kernel,
        mesh=mesh,
        out_type=jax.ShapeDtypeStruct((BATCH, _OUT_W), jnp.float32),
        scratch_types=[
            pltpu.VMEM((_BPW,), jnp.int32),
            pltpu.VMEM((_CH, _CTX_W), jnp.float32),
            pltpu.SemaphoreType.DMA,
        ],
    )
    def k(label_hbm, prefix_hbm, suffix_hbm, ctx_hbm, out_hbm, idx_v, rows_v, sem):
        wid = lax.axis_index("s") * _NC + lax.axis_index("c")
        base = wid * _BPW

        pltpu.sync_copy(
            label_hbm.at[pl.ds(base, _BPW)],
            idx_v.at[...],
        )

        pltpu.sync_copy(
            prefix_hbm.at[pl.ds(base, _BPW)],
            out_hbm.at[pl.ds(base, _BPW), pl.ds(0, _D)],
        )
        pltpu.sync_copy(
            suffix_hbm.at[pl.ds(base, _BPW)],
            out_hbm.at[pl.ds(base, _BPW), pl.ds((1 + N_CTX) * _D, _SUF_W)],
        )

        def body(c, carry):
            pltpu.async_copy(ctx_hbm.at[idx_v.at[pl.ds(c * _CH, _CH)]], rows_v, sem).wait()
            pltpu.sync_copy(
                rows_v.at[...],
                out_hbm.at[pl.ds(base + c * _CH, _CH), pl.ds(_D, _CTX_W)],
            )
            return carry

        lax.fori_loop(0, _NCHUNK, body, 0)

    return k


def kernel(label, prefix, suffix, ctx):
    label32 = label.astype(jnp.int32).reshape(BATCH)
    prefix2 = prefix.reshape(BATCH, _D)
    suffix2 = suffix.reshape(BATCH, _SUF_W)
    ctx2 = ctx.reshape(N_CLS, _CTX_W)
    out2 = _sc_prompt_kernel()(label32, prefix2, suffix2, ctx2)
    return out2.reshape(BATCH, TOT_LEN, _D)


# speedup vs baseline: 7.6201x; 7.6201x over previous
import functools

import jax
import jax.numpy as jnp
from jax import lax
from jax.experimental import pallas as pl
from jax.experimental.pallas import tpu as pltpu
from jax.experimental.pallas import tpu_sc as plsc

N_CLS = 1000
N_CTX = 16
CTX_DIM = 512
BATCH = 4096
SUF_LEN = 60
TOT_LEN = 1 + N_CTX + SUF_LEN

_D = CTX_DIM
_CTX_W = N_CTX * _D
_SUF_W = SUF_LEN * _D
_OUT_W = TOT_LEN * _D

_NC = 2
_NS = 16
_NW = _NC * _NS
_BPW = BATCH // _NW
_CH = 2
_NCHUNK = _BPW // _CH


def _sc_prompt_kernel():
    mesh = plsc.VectorSubcoreMesh(core_axis_name="c", subcore_axis_name="s")

    @functools.partial(
        pl.kernel,
        mesh=mesh,
        out_type=jax.ShapeDtypeStruct((BATCH, _OUT_W), jnp.float32),
        scratch_types=[
            pltpu.VMEM((_NCHUNK, _CH), jnp.int32),
            pltpu.VMEM((_CH, _OUT_W), jnp.float32),
            pltpu.SemaphoreType.DMA,
            pltpu.SemaphoreType.DMA,
            pltpu.SemaphoreType.DMA,
        ],
    )
    def k(label_hbm, prefix_hbm, suffix_hbm, ctx_hbm, out_hbm,
          idx_v, buf, sem_p, sem_c, sem_s):
        wid = lax.axis_index("s") * _NC + lax.axis_index("c")
        base = wid * _BPW

        pltpu.sync_copy(label_hbm.at[pl.ds(wid * _NCHUNK, _NCHUNK)], idx_v.at[...])

        def body(c, carry):
            r = base + c * _CH
            cp_p = pltpu.async_copy(
                prefix_hbm.at[pl.ds(r, _CH)],
                buf.at[:, pl.ds(0, _D)], sem_p)
            cp_c = pltpu.async_copy(
                ctx_hbm.at[idx_v.at[c]],
                buf.at[:, pl.ds(_D, _CTX_W)], sem_c)
            cp_s = pltpu.async_copy(
                suffix_hbm.at[pl.ds(r, _CH)],
                buf.at[:, pl.ds((1 + N_CTX) * _D, _SUF_W)], sem_s)
            cp_p.wait()
            cp_c.wait()
            cp_s.wait()
            pltpu.sync_copy(buf.at[...], out_hbm.at[pl.ds(r, _CH)])
            return carry

        lax.fori_loop(0, _NCHUNK, body, 0)

    return k


def kernel(label, prefix, suffix, ctx):
    label32 = label.astype(jnp.int32).reshape(BATCH // _CH, _CH)
    prefix2 = prefix.reshape(BATCH, _D)
    suffix2 = suffix.reshape(BATCH, _SUF_W)
    ctx2 = ctx.reshape(N_CLS, _CTX_W)
    out2 = _sc_prompt_kernel()(label32, prefix2, suffix2, ctx2)
    return out2.reshape(BATCH, TOT_LEN, _D)


# speedup vs baseline: 12.0152x; 1.5768x over previous
import functools

import jax
import jax.numpy as jnp
from jax import lax
from jax.experimental import pallas as pl
from jax.experimental.pallas import tpu as pltpu
from jax.experimental.pallas import tpu_sc as plsc

N_CLS = 1000
N_CTX = 16
CTX_DIM = 512
BATCH = 4096
SUF_LEN = 60
TOT_LEN = 1 + N_CTX + SUF_LEN

_D = CTX_DIM
_CTX_W = N_CTX * _D

_NC = 2
_NS = 16
_NW = _NC * _NS
_BPW = BATCH // _NW
_CH = 8
_NCHUNK = _BPW // _CH


def _sc_gather_kernel():
    mesh = plsc.VectorSubcoreMesh(core_axis_name="c", subcore_axis_name="s")

    @functools.partial(
        pl.kernel,
        mesh=mesh,
        out_type=jax.ShapeDtypeStruct((BATCH, _CTX_W), jnp.float32),
        scratch_types=[
            pltpu.VMEM((_BPW,), jnp.int32),
            pltpu.VMEM((_CH, _CTX_W), jnp.float32),
            pltpu.SemaphoreType.DMA,
        ],
    )
    def k(label_hbm, ctx_hbm, out_hbm, idx_v, buf, sem):
        wid = lax.axis_index("s") * _NC + lax.axis_index("c")
        base = wid * _BPW

        pltpu.sync_copy(label_hbm.at[pl.ds(base, _BPW)], idx_v.at[...])

        def body(c, carry):
            pltpu.async_copy(
                ctx_hbm.at[idx_v.at[pl.ds(c * _CH, _CH)]], buf, sem).wait()
            pltpu.sync_copy(buf.at[...], out_hbm.at[pl.ds(base + c * _CH, _CH)])
            return carry

        lax.fori_loop(0, _NCHUNK, body, 0)

    return k


_BB = 64


def _tc_assemble(pref_ref, gath_ref, suf_ref, out_ref):
    out_ref[:, 0:1, :] = pref_ref[...]
    out_ref[:, 1:1 + N_CTX, :] = gath_ref[...]
    out_ref[:, 1 + N_CTX:, :] = suf_ref[...]


def _tc_assemble_call(prefix, gathered3, suffix):
    return pl.pallas_call(
        _tc_assemble,
        grid=(BATCH // _BB,),
        in_specs=[
            pl.BlockSpec((_BB, 1, _D), lambda i: (i, 0, 0)),
            pl.BlockSpec((_BB, N_CTX, _D), lambda i: (i, 0, 0)),
            pl.BlockSpec((_BB, SUF_LEN, _D), lambda i: (i, 0, 0)),
        ],
        out_specs=pl.BlockSpec((_BB, TOT_LEN, _D), lambda i: (i, 0, 0)),
        out_shape=jax.ShapeDtypeStruct((BATCH, TOT_LEN, _D), jnp.float32),
    )(prefix, gathered3, suffix)


def kernel(label, prefix, suffix, ctx):
    label32 = label.astype(jnp.int32).reshape(BATCH)
    ctx2 = ctx.reshape(N_CLS, _CTX_W)
    gathered = _sc_gather_kernel()(label32, ctx2)
    gathered3 = gathered.reshape(BATCH, N_CTX, _D)
    return _tc_assemble_call(prefix, gathered3, suffix)
